# baseline (speedup 1.0000x reference)
import functools

import jax
import jax.numpy as jnp
import numpy as np
from jax import lax
from jax.experimental import pallas as pl
from jax.experimental.pallas import tpu as pltpu
from jax.experimental.pallas import tpu_sc as plsc

B, S, D = 16384, 200, 64
L = 16
NC, NS = 2, 16
NW = NC * NS
RPW = B // NW
CB = 4
CI = CB * S
NG = 8
GSZ = CI // NG
NCH = RPW // CB
NV = D // L

_MESH = plsc.VectorSubcoreMesh(
    core_axis_name="c", subcore_axis_name="s", num_cores=NC, num_subcores=NS
)


@functools.partial(
    pl.kernel,
    out_type=(
        jax.ShapeDtypeStruct((B, D), jnp.float32),
        jax.ShapeDtypeStruct((B, D), jnp.float32),
    ),
    mesh=_MESH,
    compiler_params=pltpu.CompilerParams(
        use_tc_tiling_on_sc=False, needs_layout_passes=False
    ),
    scratch_types=[
        pltpu.VMEM((NG, GSZ), jnp.int32),
        pltpu.VMEM((NG, GSZ), jnp.int32),
        pltpu.VMEM((CI, D), jnp.bfloat16),
        pltpu.VMEM((CI, D), jnp.bfloat16),
        pltpu.VMEM((RPW, D), jnp.float32),
        pltpu.SemaphoreType.DMA,
        pltpu.SemaphoreType.DMA,
        pltpu.SemaphoreType.DMA,
    ],
)
def _lookup_pool(idx_pri, idx_sec, emb_pri, emb_sec, out_pri, out_sec,
                 idx_v0, idx_v1, rows_v0, rows_v1, out_v, gsem0, gsem1,
                 isem):
    wid = lax.axis_index("s") * NC + lax.axis_index("c")
    idx_base = wid * (RPW * S // GSZ)
    row_base = wid * RPW
    idx_bufs = (idx_v0, idx_v1)
    row_bufs = (rows_v0, rows_v1)
    sems = (gsem0, gsem1)

    def do_table(idx2d, emb, out_hbm):
        def idx_copy(g, slot):
            return pltpu.make_async_copy(
                idx2d.at[pl.ds(idx_base + g * NG, NG)], idx_bufs[slot], isem
            )

        def gather_copies(slot):
            return [
                pltpu.make_async_copy(
                    emb.at[idx_bufs[slot].at[j]],
                    row_bufs[slot].at[pl.ds(j * GSZ, GSZ)],
                    sems[slot],
                )
                for j in range(NG)
            ]

        def consume(g, slot):
            rows = row_bufs[slot]
            for r in range(CB):
                def block(t, accs):
                    base = r * S + 8 * t
                    p0 = rows[base, pl.ds(0, 2 * L)]
                    p1 = rows[base, pl.ds(2 * L, 2 * L)]
                    for u in range(1, 8):
                        p0 = p0 + rows[base + u, pl.ds(0, 2 * L)]
                        p1 = p1 + rows[base + u, pl.ds(2 * L, 2 * L)]
                    new = list(accs)
                    for h, p in enumerate((p0, p1)):
                        lo, hi = plsc.unpack(
                            p, format=plsc.PackFormat.INTERLEAVED
                        )
                        new[2 * h] = new[2 * h] + lo
                        new[2 * h + 1] = new[2 * h + 1] + hi
                    return tuple(new)

                accs = lax.fori_loop(
                    0, S // 8, block,
                    tuple(jnp.zeros((L,), jnp.float32) for _ in range(NV)),
                )
                for j, a in enumerate(accs):
                    out_v[g * CB + r, pl.ds(j * L, L)] = a

        pltpu.sync_copy(idx2d.at[pl.ds(idx_base, NG)], idx_bufs[0])
        for c in gather_copies(0):
            c.start()
        idx_copy(1, 1).start()

        def pair(i, carry):
            for b in range(2):
                g = 2 * i + b
                slot = b
                nslot = (b + 1) % 2

                @pl.when(g + 1 < NCH)
                def _():
                    idx_copy(g + 1, nslot).wait()
                    for c in gather_copies(nslot):
                        c.start()

                for c in gather_copies(slot):
                    c.wait()

                @pl.when(g + 2 < NCH)
                def _():
                    idx_copy(g + 2, slot).start()

                consume(g, slot)
            return carry

        lax.fori_loop(0, NCH // 2, pair, 0)
        pltpu.sync_copy(out_v, out_hbm.at[pl.ds(row_base, RPW)])

    do_table(idx_pri, emb_pri, out_pri)
    do_table(idx_sec, emb_sec, out_sec)


def _prep_table(emb):
    v = emb.shape[0]
    t = emb.reshape(v, D // (2 * L), 2, L).transpose(0, 1, 3, 2)
    return t.reshape(v, D).astype(jnp.bfloat16)


def kernel(inputs_pri, inputs_sec, emb_pri, emb_sec):
    ip = inputs_pri.reshape(B * S // GSZ, GSZ)
    isec = inputs_sec.reshape(B * S // GSZ, GSZ)
    return _lookup_pool(ip, isec, _prep_table(emb_pri), _prep_table(emb_sec))
